# baseline (speedup 1.0000x reference)
import functools

import jax
import jax.numpy as jnp
from jax import lax
from jax.experimental import pallas as pl
from jax.experimental.pallas import tpu as pltpu
from jax.experimental.pallas import tpu_sc as plsc

B = 4
N = 4096
NC = 2
NS = 16
CHUNKS = 8
CHUNK = N // CHUNKS
IB = 32
NT = IB // 16
SUBACC = 2
LANES = 16


def _chamfer_body(
    x1x, x1y, x1z, x2x, x2y, x2z, out,
    qx, qy, qz, rx, ry, rz, colacc, dtile, idxcols, redbuf, ovec, shared,
):
    c = lax.axis_index("c")
    s = lax.axis_index("s")
    wid = s * 2 + c
    b = c * 2 + s // CHUNKS
    ch = s % CHUNKS
    base = ch * CHUNK

    iota = lax.iota(jnp.int32, LANES)
    for t in range(NT):
        for i in range(LANES):
            idxcols[pl.ds((t * LANES + i) * LANES, LANES)] = (
                iota * LANES + (t * LANES * LANES + i)
            )

    inf = jnp.full((LANES,), jnp.inf, jnp.float32)

    def initbody(v, _):
        colacc[pl.ds(v * LANES, LANES)] = inf
        return 0

    lax.fori_loop(0, N // LANES, initbody, 0)

    for src, dst in zip((x1x, x1y, x1z), (qx, qy, qz)):
        pltpu.sync_copy(src.at[b, pl.ds(base, CHUNK)], dst)
    for src, dst in zip((x2x, x2y, x2z), (rx, ry, rz)):
        pltpu.sync_copy(src.at[b], dst)

    def ibody(ib, vtotal):
        qxv = [qx[pl.ds(ib * IB + t * LANES, LANES)] for t in range(NT)]
        qyv = [qy[pl.ds(ib * IB + t * LANES, LANES)] for t in range(NT)]
        qzv = [qz[pl.ds(ib * IB + t * LANES, LANES)] for t in range(NT)]

        def jbody(jv, accs, qxv=qxv, qyv=qyv, qzv=qzv):
            rxv = rx[pl.ds(jv * LANES, LANES)]
            ryv = ry[pl.ds(jv * LANES, LANES)]
            rzv = rz[pl.ds(jv * LANES, LANES)]
            accs = list(accs)
            for l in range(LANES):
                sx = rxv[l]
                sy = ryv[l]
                sz = rzv[l]
                for t in range(NT):
                    dx = qxv[t] - sx
                    dy = qyv[t] - sy
                    dz = qzv[t] - sz
                    d = dx * dx + dy * dy + dz * dz
                    k = (l % SUBACC) * NT + t
                    accs[k] = jnp.minimum(accs[k], d)
                    dtile[pl.ds((t * LANES + l) * LANES, LANES)] = d
            g = []
            for t in range(NT):
                for i in range(LANES):
                    idxv = idxcols[pl.ds((t * LANES + i) * LANES, LANES)]
                    g.append(plsc.load_gather(dtile, [idxv]))
            while len(g) > 1:
                g = [jnp.minimum(g[2 * k], g[2 * k + 1]) for k in range(len(g) // 2)]
            cv = colacc[pl.ds(jv * LANES, LANES)]
            colacc[pl.ds(jv * LANES, LANES)] = jnp.minimum(cv, g[0])
            return tuple(accs)

        accs = lax.fori_loop(0, N // LANES, jbody, (inf,) * (SUBACC * NT))
        m = accs[0]
        for k in range(1, SUBACC * NT):
            m = jnp.minimum(m, accs[k])
        return vtotal + m

    vtotal = lax.fori_loop(0, CHUNK // IB, ibody, jnp.zeros((LANES,), jnp.float32))
    ovec[...] = vtotal * jnp.float32(1.0 / N)
    pltpu.sync_copy(ovec, out.at[wid * 2])

    pltpu.sync_copy(colacc, shared.at[s])
    plsc.subcore_barrier()
    s0 = (s // CHUNKS) * CHUNKS
    pltpu.sync_copy(shared.at[pl.ds(s0, CHUNKS), pl.ds(ch * CHUNK, CHUNK)], redbuf)

    def redbody(v, csum):
        m = redbuf[0, pl.ds(v * LANES, LANES)]
        for r in range(1, CHUNKS):
            m = jnp.minimum(m, redbuf[r, pl.ds(v * LANES, LANES)])
        return csum + m

    csum = lax.fori_loop(0, CHUNK // LANES, redbody, jnp.zeros((LANES,), jnp.float32))
    ovec[...] = csum * jnp.float32(1.0 / N)
    pltpu.sync_copy(ovec, out.at[wid * 2 + 1])


def kernel(xyz1, xyz2):
    x1 = jnp.transpose(xyz1, (2, 0, 1))
    x2 = jnp.transpose(xyz2, (2, 0, 1))

    run = functools.partial(
        pl.kernel,
        mesh=plsc.VectorSubcoreMesh(core_axis_name="c", subcore_axis_name="s"),
        compiler_params=pltpu.CompilerParams(needs_layout_passes=False),
        out_type=jax.ShapeDtypeStruct((NC * NS * 2, LANES), jnp.float32),
        scratch_types=[
            pltpu.VMEM((CHUNK,), jnp.float32),
            pltpu.VMEM((CHUNK,), jnp.float32),
            pltpu.VMEM((CHUNK,), jnp.float32),
            pltpu.VMEM((N,), jnp.float32),
            pltpu.VMEM((N,), jnp.float32),
            pltpu.VMEM((N,), jnp.float32),
            pltpu.VMEM((N,), jnp.float32),
            pltpu.VMEM((NT * LANES * LANES,), jnp.float32),
            pltpu.VMEM((NT * LANES * LANES,), jnp.int32),
            pltpu.VMEM((CHUNKS, CHUNK), jnp.float32),
            pltpu.VMEM((LANES,), jnp.float32),
            pltpu.VMEM_SHARED((NS, N), jnp.float32),
        ],
    )(_chamfer_body)
    partials = run(x1[0], x1[1], x1[2], x2[0], x2[1], x2[2])
    per_wid = partials.reshape(NS, NC, 2 * LANES).sum(axis=-1)
    return per_wid.T.reshape(NC, NC, CHUNKS).sum(axis=-1).reshape(B)


# speedup vs baseline: 1.0042x; 1.0042x over previous
import functools

import jax
import jax.numpy as jnp
from jax import lax
from jax.experimental import pallas as pl
from jax.experimental.pallas import tpu as pltpu
from jax.experimental.pallas import tpu_sc as plsc

B = 4
N = 4096
NC = 2
NS = 16
CHUNKS = 8
CHUNK = N // CHUNKS
IB = 32
NT = IB // 16
SUBACC = 2
LANES = 16


def _chamfer_body(
    x1x, x1y, x1z, x2x, x2y, x2z, out,
    qx, qy, qz, rx, ry, rz, colacc, dtile, idxcols, redbuf, ovec, shared,
):
    c = lax.axis_index("c")
    s = lax.axis_index("s")
    wid = s * 2 + c
    b = c * 2 + s // CHUNKS
    ch = s % CHUNKS
    base = ch * CHUNK

    iota = lax.iota(jnp.int32, LANES)
    for t in range(NT):
        for i in range(LANES):
            idxcols[pl.ds((t * LANES + i) * LANES, LANES)] = (
                iota * LANES + (t * LANES * LANES + i)
            )

    inf = jnp.full((LANES,), jnp.inf, jnp.float32)

    def initbody(v, _):
        colacc[pl.ds(v * LANES, LANES)] = inf
        return 0

    lax.fori_loop(0, N // LANES, initbody, 0)

    for src, dst in zip((x1x, x1y, x1z), (qx, qy, qz)):
        pltpu.sync_copy(src.at[b, pl.ds(base, CHUNK)], dst)
    for src, dst in zip((x2x, x2y, x2z), (rx, ry, rz)):
        pltpu.sync_copy(src.at[b], dst)

    def ibody(ib, vtotal):
        qxv = [qx[pl.ds(ib * IB + t * LANES, LANES)] for t in range(NT)]
        qyv = [qy[pl.ds(ib * IB + t * LANES, LANES)] for t in range(NT)]
        qzv = [qz[pl.ds(ib * IB + t * LANES, LANES)] for t in range(NT)]

        def jbody(jv, accs, qxv=qxv, qyv=qyv, qzv=qzv):
            rxv = rx[pl.ds(jv * LANES, LANES)]
            ryv = ry[pl.ds(jv * LANES, LANES)]
            rzv = rz[pl.ds(jv * LANES, LANES)]
            accs = list(accs)
            for l in range(LANES):
                sx = rxv[l]
                sy = ryv[l]
                sz = rzv[l]
                for t in range(NT):
                    dx = qxv[t] - sx
                    dy = qyv[t] - sy
                    dz = qzv[t] - sz
                    d = dx * dx + dy * dy + dz * dz
                    k = (l % SUBACC) * NT + t
                    accs[k] = jnp.minimum(accs[k], d)
                    dtile[pl.ds((t * LANES + l) * LANES, LANES)] = d
            g = []
            for t in range(NT):
                for i in range(LANES):
                    idxv = idxcols[pl.ds((t * LANES + i) * LANES, LANES)]
                    g.append(plsc.load_gather(dtile, [idxv]))
            while len(g) > 1:
                g = [jnp.minimum(g[2 * k], g[2 * k + 1]) for k in range(len(g) // 2)]
            cv = colacc[pl.ds(jv * LANES, LANES)]
            colacc[pl.ds(jv * LANES, LANES)] = jnp.minimum(cv, g[0])
            return tuple(accs)

        accs = lax.fori_loop(0, N // LANES, jbody, (inf,) * (SUBACC * NT))
        blocksum = None
        for t in range(NT):
            m = accs[t]
            for k in range(1, SUBACC):
                m = jnp.minimum(m, accs[k * NT + t])
            blocksum = m if blocksum is None else blocksum + m
        return vtotal + blocksum

    vtotal = lax.fori_loop(0, CHUNK // IB, ibody, jnp.zeros((LANES,), jnp.float32))
    ovec[...] = vtotal * jnp.float32(1.0 / N)
    pltpu.sync_copy(ovec, out.at[wid * 2])

    pltpu.sync_copy(colacc, shared.at[s])
    plsc.subcore_barrier()
    s0 = (s // CHUNKS) * CHUNKS
    pltpu.sync_copy(shared.at[pl.ds(s0, CHUNKS), pl.ds(ch * CHUNK, CHUNK)], redbuf)

    def redbody(v, csum):
        m = redbuf[0, pl.ds(v * LANES, LANES)]
        for r in range(1, CHUNKS):
            m = jnp.minimum(m, redbuf[r, pl.ds(v * LANES, LANES)])
        return csum + m

    csum = lax.fori_loop(0, CHUNK // LANES, redbody, jnp.zeros((LANES,), jnp.float32))
    ovec[...] = csum * jnp.float32(1.0 / N)
    pltpu.sync_copy(ovec, out.at[wid * 2 + 1])


def kernel(xyz1, xyz2):
    x1 = jnp.transpose(xyz1, (2, 0, 1))
    x2 = jnp.transpose(xyz2, (2, 0, 1))

    run = functools.partial(
        pl.kernel,
        mesh=plsc.VectorSubcoreMesh(core_axis_name="c", subcore_axis_name="s"),
        compiler_params=pltpu.CompilerParams(needs_layout_passes=False),
        out_type=jax.ShapeDtypeStruct((NC * NS * 2, LANES), jnp.float32),
        scratch_types=[
            pltpu.VMEM((CHUNK,), jnp.float32),
            pltpu.VMEM((CHUNK,), jnp.float32),
            pltpu.VMEM((CHUNK,), jnp.float32),
            pltpu.VMEM((N,), jnp.float32),
            pltpu.VMEM((N,), jnp.float32),
            pltpu.VMEM((N,), jnp.float32),
            pltpu.VMEM((N,), jnp.float32),
            pltpu.VMEM((NT * LANES * LANES,), jnp.float32),
            pltpu.VMEM((NT * LANES * LANES,), jnp.int32),
            pltpu.VMEM((CHUNKS, CHUNK), jnp.float32),
            pltpu.VMEM((LANES,), jnp.float32),
            pltpu.VMEM_SHARED((NS, N), jnp.float32),
        ],
    )(_chamfer_body)
    partials = run(x1[0], x1[1], x1[2], x2[0], x2[1], x2[2])
    per_wid = partials.reshape(NS, NC, 2 * LANES).sum(axis=-1)
    return per_wid.T.reshape(NC, NC, CHUNKS).sum(axis=-1).reshape(B)


# speedup vs baseline: 2.9425x; 2.9302x over previous
import functools

import jax
import jax.numpy as jnp
from jax import lax
from jax.experimental import pallas as pl
from jax.experimental.pallas import tpu as pltpu
from jax.experimental.pallas import tpu_sc as plsc

B = 4
N = 4096
NC = 2
NS = 16
NW = NC * NS
SC_CHUNK = N // NW
IB = 32
NT = IB // 16
SUBACC = 2
LANES = 16
SLICE = N // NS
TN = 512


def _sc_body(
    x1x, x1y, x1z, x2x, x2y, x2z, rowparts, colparts,
    qx, qy, qz, rx, ry, rz, colacc, dtile, idxcols, redbuf, mslice, ovec, shared,
):
    c = lax.axis_index("c")
    s = lax.axis_index("s")
    wid = s * 2 + c
    base = wid * SC_CHUNK

    iota = lax.iota(jnp.int32, LANES)
    for t in range(NT):
        for i in range(LANES):
            idxcols[pl.ds((t * LANES + i) * LANES, LANES)] = (
                iota * LANES + (t * LANES * LANES + i)
            )

    inf = jnp.full((LANES,), jnp.inf, jnp.float32)

    def initbody(v, _):
        colacc[pl.ds(v * LANES, LANES)] = inf
        return 0

    lax.fori_loop(0, N // LANES, initbody, 0)

    for src, dst in zip((x1x, x1y, x1z), (qx, qy, qz)):
        pltpu.sync_copy(src.at[pl.ds(base, SC_CHUNK)], dst)
    for src, dst in zip((x2x, x2y, x2z), (rx, ry, rz)):
        pltpu.sync_copy(src, dst)

    def ibody(ib, vtotal):
        qxv = [qx[pl.ds(ib * IB + t * LANES, LANES)] for t in range(NT)]
        qyv = [qy[pl.ds(ib * IB + t * LANES, LANES)] for t in range(NT)]
        qzv = [qz[pl.ds(ib * IB + t * LANES, LANES)] for t in range(NT)]

        def jbody(jv, accs, qxv=qxv, qyv=qyv, qzv=qzv):
            rxv = rx[pl.ds(jv * LANES, LANES)]
            ryv = ry[pl.ds(jv * LANES, LANES)]
            rzv = rz[pl.ds(jv * LANES, LANES)]
            accs = list(accs)
            for l in range(LANES):
                sx = rxv[l]
                sy = ryv[l]
                sz = rzv[l]
                for t in range(NT):
                    dx = qxv[t] - sx
                    dy = qyv[t] - sy
                    dz = qzv[t] - sz
                    d = dx * dx + dy * dy + dz * dz
                    k = (l % SUBACC) * NT + t
                    accs[k] = jnp.minimum(accs[k], d)
                    dtile[pl.ds((t * LANES + l) * LANES, LANES)] = d
            g = []
            for t in range(NT):
                for i in range(LANES):
                    idxv = idxcols[pl.ds((t * LANES + i) * LANES, LANES)]
                    g.append(plsc.load_gather(dtile, [idxv]))
            while len(g) > 1:
                g = [jnp.minimum(g[2 * k], g[2 * k + 1]) for k in range(len(g) // 2)]
            cv = colacc[pl.ds(jv * LANES, LANES)]
            colacc[pl.ds(jv * LANES, LANES)] = jnp.minimum(cv, g[0])
            return tuple(accs)

        accs = lax.fori_loop(0, N // LANES, jbody, (inf,) * (SUBACC * NT))
        blocksum = None
        for t in range(NT):
            m = accs[t]
            for k in range(1, SUBACC):
                m = jnp.minimum(m, accs[k * NT + t])
            blocksum = m if blocksum is None else blocksum + m
        return vtotal + blocksum

    vtotal = lax.fori_loop(0, SC_CHUNK // IB, ibody, jnp.zeros((LANES,), jnp.float32))
    ovec[...] = vtotal * jnp.float32(1.0 / N)
    pltpu.sync_copy(ovec, rowparts.at[wid])

    pltpu.sync_copy(colacc, shared.at[s])
    plsc.subcore_barrier()
    pltpu.sync_copy(shared.at[pl.ds(0, NS), pl.ds(s * SLICE, SLICE)], redbuf)

    def redbody(v, _):
        m = redbuf[0, pl.ds(v * LANES, LANES)]
        for r in range(1, NS):
            m = jnp.minimum(m, redbuf[r, pl.ds(v * LANES, LANES)])
        mslice[pl.ds(v * LANES, LANES)] = m
        return 0

    lax.fori_loop(0, SLICE // LANES, redbody, 0)
    pltpu.sync_copy(mslice, colparts.at[c, pl.ds(s * SLICE, SLICE)])


def _tc_main_body(x1_ref, x2_ref, colmin_ref, rowagg_ref):
    i = pl.program_id(1)
    a = x1_ref[0]
    b = x2_ref[0]
    qn = a[0] * a[0] + a[1] * a[1] + a[2] * a[2]
    rn = b[0] * b[0] + b[1] * b[1] + b[2] * b[2]
    e = lax.dot_general(
        a * jnp.float32(-2.0), b, (((0,), (0,)), ((), ())),
        precision=lax.Precision.HIGHEST,
        preferred_element_type=jnp.float32,
    )
    rowpart = jnp.sum(jnp.min(e + rn[None, :], axis=1) + qn) * jnp.float32(1.0 / N)
    cmin = jnp.min(e + qn[:, None], axis=0, keepdims=True)

    @pl.when(i == 0)
    def _():
        rowagg_ref[...] = jnp.full((1, 1, 128), rowpart, jnp.float32)
        colmin_ref[...] = cmin[None]

    @pl.when(i > 0)
    def _():
        rowagg_ref[...] = rowagg_ref[...] + rowpart
        colmin_ref[...] = jnp.minimum(colmin_ref[...], cmin[None])


def _tc_merge_body(
    colmin_ref, rowagg_ref, x2_ref, sccol_ref, scrow_ref, out_ref
):
    b = x2_ref[...]
    rn = b[:, 0, :] ** 2 + b[:, 1, :] ** 2 + b[:, 2, :] ** 2
    colmean = jnp.mean(colmin_ref[:, 0, :] + rn, axis=1)
    tc_out = rowagg_ref[:, 0, 0] + colmean
    sc_col = jnp.minimum(sccol_ref[0, :], sccol_ref[1, :])
    sc_out = jnp.sum(scrow_ref[...]) + jnp.mean(sc_col)
    res = jnp.where(lax.iota(jnp.int32, B) == 0, sc_out, tc_out)
    out_ref[...] = jnp.broadcast_to(res[:, None], (B, 128))


def kernel(xyz1, xyz2):
    x1 = jnp.transpose(xyz1, (2, 0, 1))
    x2 = jnp.transpose(xyz2, (2, 0, 1))
    x1t = jnp.transpose(xyz1, (0, 2, 1))
    x2t = jnp.transpose(xyz2, (0, 2, 1))

    sc_run = functools.partial(
        pl.kernel,
        mesh=plsc.VectorSubcoreMesh(core_axis_name="c", subcore_axis_name="s"),
        compiler_params=pltpu.CompilerParams(needs_layout_passes=False),
        out_type=(
            jax.ShapeDtypeStruct((NW, LANES), jnp.float32),
            jax.ShapeDtypeStruct((NC, N), jnp.float32),
        ),
        scratch_types=[
            pltpu.VMEM((SC_CHUNK,), jnp.float32),
            pltpu.VMEM((SC_CHUNK,), jnp.float32),
            pltpu.VMEM((SC_CHUNK,), jnp.float32),
            pltpu.VMEM((N,), jnp.float32),
            pltpu.VMEM((N,), jnp.float32),
            pltpu.VMEM((N,), jnp.float32),
            pltpu.VMEM((N,), jnp.float32),
            pltpu.VMEM((NT * LANES * LANES,), jnp.float32),
            pltpu.VMEM((NT * LANES * LANES,), jnp.int32),
            pltpu.VMEM((NS, SLICE), jnp.float32),
            pltpu.VMEM((SLICE,), jnp.float32),
            pltpu.VMEM((LANES,), jnp.float32),
            pltpu.VMEM_SHARED((NS, N), jnp.float32),
        ],
    )(_sc_body)
    rowparts, colparts = sc_run(
        x1[0][0], x1[1][0], x1[2][0], x2[0][0], x2[1][0], x2[2][0]
    )

    colmin, rowagg = pl.pallas_call(
        _tc_main_body,
        grid=(B - 1, N // TN),
        in_specs=[
            pl.BlockSpec((1, 3, TN), lambda pb, i: (pb + 1, 0, i)),
            pl.BlockSpec((1, 3, N), lambda pb, i: (pb + 1, 0, 0)),
        ],
        out_specs=[
            pl.BlockSpec((1, 1, N), lambda pb, i: (pb + 1, 0, 0)),
            pl.BlockSpec((1, 1, 128), lambda pb, i: (pb + 1, 0, 0)),
        ],
        out_shape=[
            jax.ShapeDtypeStruct((B, 1, N), jnp.float32),
            jax.ShapeDtypeStruct((B, 1, 128), jnp.float32),
        ],
    )(x1t, x2t)

    out = pl.pallas_call(
        _tc_merge_body,
        out_shape=jax.ShapeDtypeStruct((B, 128), jnp.float32),
    )(colmin, rowagg, x2t, colparts, rowparts)
    return out[:, 0]
